# Initial kernel scaffold; baseline (speedup 1.0000x reference)
import jax
import jax.numpy as jnp
from jax.experimental import pallas as pl


def kernel(positions, node_attrs, edge_index, batch, head, shifts, ptr, W_ae, W_emb, W_up0, W_r1_0, W_r2_0, W_down0, W_sc0, W_prod0, W_read0, W_up1, W_r1_1, W_r2_1, W_down1, W_sc1, W_prod1, W_read1):
    raise NotImplementedError("write your pallas kernel here")


# speedup vs baseline: 7.1584x; 7.1584x over previous
import functools

import jax
import jax.numpy as jnp
from jax import lax
from jax.experimental import pallas as pl
from jax.experimental.pallas import tpu as pltpu
from jax.experimental.pallas import tpu_sc as plsc

N = 10000
E = 320000
NEL = 128
H = 64
NSH = 4
NB = 8
RMAX = 5.0
G = 16

NC = 2
NS = 16
NW = NC * NS
CHUNK = 128
NCHUNKS = E // CHUNK

EB = 5000
NBLK = 2000

_mesh = functools.partial(
    plsc.VectorSubcoreMesh, core_axis_name="c", subcore_axis_name="s",
    num_cores=NC, num_subcores=NS)


def _silu(x):
    return x * (1.0 / (1.0 + jnp.exp(-x)))


def _node_init_body(na_ref, b_ref, wae_ref, wemb_ref, wup_ref,
                    nf_ref, h_ref, e_ref):
    na = na_ref[...]
    nf = jnp.dot(na, wemb_ref[...], preferred_element_type=jnp.float32)
    nf_ref[...] = nf
    h_ref[...] = jnp.dot(nf, wup_ref[...], preferred_element_type=jnp.float32)
    ne0 = jnp.dot(na, wae_ref[...], preferred_element_type=jnp.float32)
    onehot = (b_ref[...] == lax.broadcasted_iota(jnp.int32, (1, G), 1)
              ).astype(jnp.float32)
    e_ref[...] = jnp.sum(onehot * ne0, axis=0, keepdims=True)


def _node_init(node_attrs, batch2d, W_ae, W_emb, W_up0):
    return pl.pallas_call(
        _node_init_body,
        out_shape=(
            jax.ShapeDtypeStruct((N, H), jnp.float32),
            jax.ShapeDtypeStruct((N, H), jnp.float32),
            jax.ShapeDtypeStruct((1, G), jnp.float32),
        ),
    )(node_attrs, batch2d, W_ae, W_emb, W_up0)


def _gather_pos_body(pos_hbm, snd_hbm, rcv_hbm, ps_hbm, pr_hbm,
                     idx_s, idx_r, rows_s, rows_r, pos_sh, sem):
    c = lax.axis_index("c")
    s = lax.axis_index("s")
    wid = s * NC + c

    @pl.when(s == 0)
    def _():
        pltpu.sync_copy(pos_hbm, pos_sh)

    plsc.subcore_barrier()
    nj = (NCHUNKS - wid + NW - 1) // NW

    def body(j, carry):
        base = (wid + j * NW) * CHUNK
        pltpu.sync_copy(snd_hbm.at[pl.ds(base, CHUNK)], idx_s)
        pltpu.sync_copy(rcv_hbm.at[pl.ds(base, CHUNK)], idx_r)
        cp1 = pltpu.async_copy(pos_sh.at[idx_s], rows_s, sem)
        cp2 = pltpu.async_copy(pos_sh.at[idx_r], rows_r, sem)
        cp1.wait()
        cp2.wait()
        pltpu.sync_copy(rows_s, ps_hbm.at[pl.ds(base, CHUNK)])
        pltpu.sync_copy(rows_r, pr_hbm.at[pl.ds(base, CHUNK)])
        return carry

    lax.fori_loop(0, nj, body, 0)


def _gather_pos(pos16, sender, receiver):
    return pl.kernel(
        _gather_pos_body,
        out_type=(
            jax.ShapeDtypeStruct((E, 16), jnp.float32),
            jax.ShapeDtypeStruct((E, 16), jnp.float32),
        ),
        mesh=_mesh(),
        scratch_types=[
            pltpu.VMEM((CHUNK,), jnp.int32),
            pltpu.VMEM((CHUNK,), jnp.int32),
            pltpu.VMEM((CHUNK, 16), jnp.float32),
            pltpu.VMEM((CHUNK, 16), jnp.float32),
            pltpu.VMEM_SHARED((N, 16), jnp.float32),
            pltpu.SemaphoreType.DMA,
        ],
    )(pos16, sender, receiver)


def _gather_h_body(h_hbm, snd_hbm, hs_hbm, idx_s, rows, h_sh, sem):
    c = lax.axis_index("c")
    s = lax.axis_index("s")
    wid = s * NC + c

    @pl.when(s == 0)
    def _():
        pltpu.sync_copy(h_hbm, h_sh)

    plsc.subcore_barrier()
    nj = (NCHUNKS - wid + NW - 1) // NW

    def body(j, carry):
        base = (wid + j * NW) * CHUNK
        pltpu.sync_copy(snd_hbm.at[pl.ds(base, CHUNK)], idx_s)
        pltpu.async_copy(h_sh.at[idx_s], rows, sem).wait()
        pltpu.sync_copy(rows, hs_hbm.at[pl.ds(base, CHUNK)])
        return carry

    lax.fori_loop(0, nj, body, 0)


def _gather_h(h, sender):
    return pl.kernel(
        _gather_h_body,
        out_type=jax.ShapeDtypeStruct((E, H), jnp.float32),
        mesh=_mesh(),
        scratch_types=[
            pltpu.VMEM((CHUNK,), jnp.int32),
            pltpu.VMEM((CHUNK, H), jnp.float32),
            pltpu.VMEM_SHARED((N, H), jnp.float32),
            pltpu.SemaphoreType.DMA,
        ],
    )(h, sender)


NRB = 128
NBLOCKS = N // NRB
NTAIL = N - NBLOCKS * NRB


def _scatter_body(q_hbm, rcv_hbm, rnk_hbm, nr_hbm, zero_hbm, out_hbm,
                  idx_r, rnk, idx2, nrv, rows, acc, sem):
    c = lax.axis_index("c")
    s = lax.axis_index("s")
    nb = (NBLOCKS - s + NS - 1) // NS

    def zbody(b, carry):
        r = (s + b * NS) * NRB
        pltpu.sync_copy(zero_hbm, acc.at[pl.ds(r, NRB)])
        return carry

    lax.fori_loop(0, nb, zbody, 0)

    @pl.when(s == NS - 1)
    def _():
        pltpu.sync_copy(zero_hbm.at[pl.ds(0, NTAIL)],
                        acc.at[pl.ds(NBLOCKS * NRB, NTAIL)])

    plsc.subcore_barrier()

    half = NCHUNKS // NC

    @pl.when(s == 0)
    def _():
        lane = lax.broadcasted_iota(jnp.int32, (16,), 0)
        pltpu.sync_copy(nr_hbm, nrv.at[pl.ds(0, NCHUNKS)])

        def body(j, carry):
            base = (c * half + j) * CHUNK
            pltpu.sync_copy(rcv_hbm.at[pl.ds(base, CHUNK)], idx_r)
            pltpu.sync_copy(rnk_hbm.at[pl.ds(base, CHUNK)], rnk)
            pltpu.sync_copy(q_hbm.at[pl.ds(base, CHUNK)], rows)
            nrounds = nrv[pl.ds(c * half + j, 16)][0]

            def rbody(r, c2):
                for v in range(CHUNK // 16):
                    rv = rnk[pl.ds(v * 16, 16)]
                    iv = idx_r[pl.ds(v * 16, 16)]
                    garb = N + v * 16 + lane
                    idx2[pl.ds(v * 16, 16)] = jnp.where(rv == r, iv, garb)
                pltpu.sync_copy(rows, acc.at[idx2], add=True)
                return c2

            lax.fori_loop(0, nrounds, rbody, 0)
            return carry

        lax.fori_loop(0, half, body, 0)

    plsc.subcore_barrier()

    def obody(b, carry):
        r = (s + b * NS) * NRB
        pltpu.sync_copy(acc.at[pl.ds(r, NRB)], out_hbm.at[c, pl.ds(r, NRB)])
        return carry

    lax.fori_loop(0, nb, obody, 0)

    @pl.when(s == NS - 1)
    def _():
        pltpu.sync_copy(acc.at[pl.ds(NBLOCKS * NRB, NTAIL)],
                        out_hbm.at[c, pl.ds(NBLOCKS * NRB, NTAIL)])


def _sc_scatter(q, receiver, rank, nrounds, zeros_tile):
    return pl.kernel(
        _scatter_body,
        out_type=jax.ShapeDtypeStruct((NC, N, 2 * H), jnp.float32),
        mesh=_mesh(),
        scratch_types=[
            pltpu.VMEM((CHUNK,), jnp.int32),
            pltpu.VMEM((CHUNK,), jnp.int32),
            pltpu.VMEM((CHUNK,), jnp.int32),
            pltpu.VMEM((NCHUNKS + 16,), jnp.int32),
            pltpu.VMEM((CHUNK, 2 * H), jnp.float32),
            pltpu.VMEM_SHARED((N + CHUNK, 2 * H), jnp.float32),
            pltpu.SemaphoreType.DMA,
        ],
    )(q, receiver, rank, nrounds, zeros_tile)


EB2 = 2560


def _rank_body(rcv_ref, rank_ref, nr_ref):
    row = lax.broadcasted_iota(jnp.int32, (CHUNK, CHUNK), 0)
    col = lax.broadcasted_iota(jnp.int32, (CHUNK, CHUNK), 1)
    earlier = col < row
    for c in range(EB2 // CHUNK):
        rc = rcv_ref[pl.ds(c * CHUNK, CHUNK), :]
        rt = jnp.reshape(rc, (1, CHUNK))
        eq = (rc == rt) & earlier
        cnt = jnp.sum(eq.astype(jnp.int32), axis=1, keepdims=True)
        rank_ref[pl.ds(c * CHUNK, CHUNK), :] = cnt
        nr_ref[0, pl.ds(c, 1), :] = jnp.max(cnt, axis=0, keepdims=True) + 1


def _tc_rank(receiver2d):
    return pl.pallas_call(
        _rank_body,
        grid=(E // EB2,),
        in_specs=[pl.BlockSpec((EB2, 1), lambda i: (i, 0))],
        out_specs=[
            pl.BlockSpec((EB2, 1), lambda i: (i, 0)),
            pl.BlockSpec((1, EB2 // CHUNK, 1), lambda i: (i, 0, 0)),
        ],
        out_shape=(
            jax.ShapeDtypeStruct((E, 1), jnp.int32),
            jax.ShapeDtypeStruct((E // EB2, EB2 // CHUNK, 1), jnp.int32),
        ),
    )(receiver2d)


def _edge_pre_body(ps_ref, pr_ref, shf_ref, wr10_ref, wr20_ref,
                   wr11_ref, wr21_ref, sh_ref, rw0_ref, rw1_ref):
    eps = 1e-9
    vec = pr_ref[...][:, :3] - ps_ref[...][:, :3] + shf_ref[...]
    r = jnp.sqrt(jnp.sum(vec * vec, axis=1, keepdims=True))
    v = vec / (r + eps)
    s3 = 3.0 ** 0.5
    ones = jnp.ones_like(r)
    sh_ref[...] = jnp.concatenate(
        [ones, s3 * v[:, 0:1], s3 * v[:, 1:2], s3 * v[:, 2:3]], axis=1)

    x = (jnp.pi / RMAX) * jnp.minimum(r, RMAX)
    uh = 0.5 * x

    def _cosp(t):
        t2 = t * t
        return 1.0 + t2 * (-0.5 + t2 * (1.0 / 24.0 + t2 * (-1.0 / 720.0
                   + t2 * (1.0 / 40320.0 + t2 * (-1.0 / 3628800.0)))))

    sin_h = _cosp(uh - (jnp.pi / 2))
    cos_h = _cosp(uh)
    s1 = 2.0 * sin_h * cos_h
    c1 = 1.0 - 2.0 * sin_h * sin_h
    u = r / RMAX
    u6 = u * u * u * u * u * u
    env = 1.0 - 28.0 * u6 + 48.0 * u6 * u - 21.0 * u6 * u * u
    env = env * (u < 1.0).astype(jnp.float32)
    w = (((2.0 / RMAX) ** 0.5) / (r + eps)) * env

    pre0 = jnp.zeros((EB, H), jnp.float32)
    pre1 = jnp.zeros((EB, H), jnp.float32)
    sprev = jnp.zeros_like(s1)
    scur = s1
    tc = 2.0 * c1
    for nn in range(NB):
        efn = w * scur
        pre0 = pre0 + efn * wr10_ref[nn:nn + 1, :]
        pre1 = pre1 + efn * wr11_ref[nn:nn + 1, :]
        snew = tc * scur - sprev
        sprev, scur = scur, snew

    rw0_ref[...] = jnp.dot(_silu(pre0), wr20_ref[...],
                           preferred_element_type=jnp.float32)
    rw1_ref[...] = jnp.dot(_silu(pre1), wr21_ref[...],
                           preferred_element_type=jnp.float32)


def _edge_pre(ps, pr, shifts, W_r1_0, W_r2_0, W_r1_1, W_r2_1):
    nsteps = E // EB
    wspec = pl.BlockSpec(index_map=lambda i: (0, 0))
    return pl.pallas_call(
        _edge_pre_body,
        grid=(nsteps,),
        in_specs=[
            pl.BlockSpec((EB, 16), lambda i: (i, 0)),
            pl.BlockSpec((EB, 16), lambda i: (i, 0)),
            pl.BlockSpec((EB, 3), lambda i: (i, 0)),
            wspec, wspec, wspec, wspec,
        ],
        out_specs=[
            pl.BlockSpec((EB, NSH), lambda i: (i, 0)),
            pl.BlockSpec((EB, H), lambda i: (i, 0)),
            pl.BlockSpec((EB, H), lambda i: (i, 0)),
        ],
        out_shape=(
            jax.ShapeDtypeStruct((E, NSH), jnp.float32),
            jax.ShapeDtypeStruct((E, H), jnp.float32),
            jax.ShapeDtypeStruct((E, H), jnp.float32),
        ),
    )(ps, pr, shifts, W_r1_0, W_r2_0, W_r1_1, W_r2_1)


def _msg_body(hs_ref, rw_ref, sh_ref, wd_ref, out_ref):
    ev = hs_ref[...] * rw_ref[...]
    sh = sh_ref[...]
    wd = wd_ref[...]
    q = jnp.zeros((EB, H), jnp.float32)
    for s in range(NSH):
        t = jnp.dot(ev, wd[s * H:(s + 1) * H, :],
                    preferred_element_type=jnp.float32)
        q = q + t * sh[:, s:s + 1]
    out_ref[:, :H] = q
    out_ref[:, H:] = jnp.zeros((EB, H), jnp.float32)


def _tc_msg(hs, rw, sh, W_down):
    nsteps = E // EB
    return pl.pallas_call(
        _msg_body,
        grid=(nsteps,),
        in_specs=[
            pl.BlockSpec((EB, H), lambda i: (i, 0)),
            pl.BlockSpec((EB, H), lambda i: (i, 0)),
            pl.BlockSpec((EB, NSH), lambda i: (i, 0)),
            pl.BlockSpec(index_map=lambda i: (0, 0)),
        ],
        out_specs=pl.BlockSpec((EB, 2 * H), lambda i: (i, 0)),
        out_shape=jax.ShapeDtypeStruct((E, 2 * H), jnp.float32),
    )(hs, rw, sh, W_down)


def _layer_body(a0_ref, a1_ref, nf_ref, b_ref, eprev_ref,
                wsc_ref, wp_ref, wread_ref, wupn_ref,
                nf2_ref, h_ref, e_ref):
    msg = a0_ref[0][:, :H] + a1_ref[0][:, :H]
    nf2 = (jnp.dot(_silu(msg), wp_ref[...], preferred_element_type=jnp.float32)
           + jnp.dot(nf_ref[...], wsc_ref[...], preferred_element_type=jnp.float32))
    nf2_ref[...] = nf2
    h_ref[...] = jnp.dot(nf2, wupn_ref[...], preferred_element_type=jnp.float32)
    es = jnp.dot(nf2, wread_ref[...], preferred_element_type=jnp.float32)
    onehot = (b_ref[...] == lax.broadcasted_iota(jnp.int32, (1, G), 1)
              ).astype(jnp.float32)
    contrib = jnp.sum(onehot * es, axis=0, keepdims=True)

    @pl.when(pl.program_id(0) == 0)
    def _():
        e_ref[...] = eprev_ref[...] + contrib

    @pl.when(pl.program_id(0) > 0)
    def _():
        e_ref[...] = e_ref[...] + contrib


def _tc_layer(agg, nf, batch2d, e_prev, W_sc, W_prod, W_read, W_up_next):
    nsteps = N // NBLK
    wspec = pl.BlockSpec(index_map=lambda i: (0, 0))
    return pl.pallas_call(
        _layer_body,
        grid=(nsteps,),
        in_specs=[
            pl.BlockSpec((1, NBLK, 2 * H), lambda i: (0, i, 0)),
            pl.BlockSpec((1, NBLK, 2 * H), lambda i: (1, i, 0)),
            pl.BlockSpec((NBLK, H), lambda i: (i, 0)),
            pl.BlockSpec((NBLK, 1), lambda i: (i, 0)),
            wspec, wspec, wspec, wspec, wspec,
        ],
        out_specs=[
            pl.BlockSpec((NBLK, H), lambda i: (i, 0)),
            pl.BlockSpec((NBLK, H), lambda i: (i, 0)),
            pl.BlockSpec((1, G), lambda i: (0, 0)),
        ],
        out_shape=(
            jax.ShapeDtypeStruct((N, H), jnp.float32),
            jax.ShapeDtypeStruct((N, H), jnp.float32),
            jax.ShapeDtypeStruct((1, G), jnp.float32),
        ),
    )(agg, agg, nf, batch2d, e_prev, W_sc, W_prod, W_read, W_up_next)


def kernel(positions, node_attrs, edge_index, batch, head, shifts, ptr,
           W_ae, W_emb, W_up0, W_r1_0, W_r2_0, W_down0, W_sc0, W_prod0,
           W_read0, W_up1, W_r1_1, W_r2_1, W_down1, W_sc1, W_prod1, W_read1):
    sender = edge_index[0].astype(jnp.int32)
    receiver = edge_index[1].astype(jnp.int32)
    pos16 = jnp.pad(positions, ((0, 0), (0, 13)))
    batch2d = batch.reshape(N, 1).astype(jnp.int32)
    zeros_tile = jnp.zeros((NRB, 2 * H), jnp.float32)

    nf, h, e = _node_init(node_attrs, batch2d, W_ae, W_emb, W_up0)
    ps, pr = _gather_pos(pos16, sender, receiver)
    sh, rw0, rw1 = _edge_pre(ps, pr, shifts, W_r1_0, W_r2_0, W_r1_1, W_r2_1)
    rank2d, nrounds2d = _tc_rank(receiver.reshape(E, 1))
    rank = rank2d.reshape(E)
    nrounds = nrounds2d.reshape(NCHUNKS)

    layer_params = [
        (rw0, W_down0, W_sc0, W_prod0, W_read0, W_up1),
        (rw1, W_down1, W_sc1, W_prod1, W_read1, W_up1),
    ]
    for rw, W_down, W_sc, W_prod, W_read, W_up_next in layer_params:
        hs = _gather_h(h, sender)
        q = _tc_msg(hs, rw, sh, W_down)
        agg = _sc_scatter(q, receiver, rank, nrounds, zeros_tile)
        nf, h, e = _tc_layer(agg, nf, batch2d, e, W_sc, W_prod,
                             W_read, W_up_next)
    return e.reshape(G)


# speedup vs baseline: 27.9623x; 3.9062x over previous
import functools

import jax
import jax.numpy as jnp
from jax import lax
from jax.experimental import pallas as pl
from jax.experimental.pallas import tpu as pltpu
from jax.experimental.pallas import tpu_sc as plsc

N = 10000
E = 320000
NEL = 128
H = 64
NSH = 4
NB = 8
RMAX = 5.0
G = 16

NC = 2
NS = 16
NW = NC * NS
CHUNK = 128
NCHUNKS = E // CHUNK

EB = 5000
NBLK = 2000

_mesh = functools.partial(
    plsc.VectorSubcoreMesh, core_axis_name="c", subcore_axis_name="s",
    num_cores=NC, num_subcores=NS)


def _silu(x):
    return x * (1.0 / (1.0 + jnp.exp(-x)))


def _node_init_body(na_ref, b_ref, wae_ref, wemb_ref, wup_ref,
                    nf_ref, h_ref, e_ref):
    na = na_ref[...]
    nf = jnp.dot(na, wemb_ref[...], preferred_element_type=jnp.float32)
    nf_ref[...] = nf
    h_ref[...] = jnp.dot(nf, wup_ref[...], preferred_element_type=jnp.float32)
    ne0 = jnp.dot(na, wae_ref[...], preferred_element_type=jnp.float32)
    onehot = (b_ref[...] == lax.broadcasted_iota(jnp.int32, (1, G), 1)
              ).astype(jnp.float32)
    e_ref[...] = jnp.sum(onehot * ne0, axis=0, keepdims=True)


def _node_init(node_attrs, batch2d, W_ae, W_emb, W_up0):
    return pl.pallas_call(
        _node_init_body,
        out_shape=(
            jax.ShapeDtypeStruct((N, H), jnp.float32),
            jax.ShapeDtypeStruct((N, H), jnp.float32),
            jax.ShapeDtypeStruct((1, G), jnp.float32),
        ),
    )(node_attrs, batch2d, W_ae, W_emb, W_up0)


def _gather_pos_body(pos_hbm, snd_hbm, rcv_hbm, ps_hbm, pr_hbm,
                     idx_s, idx_r, rows_s, rows_r, pos_sh, sem):
    c = lax.axis_index("c")
    s = lax.axis_index("s")
    wid = s * NC + c

    @pl.when(s == 0)
    def _():
        pltpu.sync_copy(pos_hbm, pos_sh)

    plsc.subcore_barrier()
    nj = (NCHUNKS - wid + NW - 1) // NW

    def body(j, carry):
        base = (wid + j * NW) * CHUNK
        pltpu.sync_copy(snd_hbm.at[pl.ds(base, CHUNK)], idx_s)
        pltpu.sync_copy(rcv_hbm.at[pl.ds(base, CHUNK)], idx_r)
        cp1 = pltpu.async_copy(pos_sh.at[idx_s], rows_s, sem)
        cp2 = pltpu.async_copy(pos_sh.at[idx_r], rows_r, sem)
        cp1.wait()
        cp2.wait()
        pltpu.sync_copy(rows_s, ps_hbm.at[pl.ds(base, CHUNK)])
        pltpu.sync_copy(rows_r, pr_hbm.at[pl.ds(base, CHUNK)])
        return carry

    lax.fori_loop(0, nj, body, 0)


def _gather_pos(pos16, sender, receiver):
    return pl.kernel(
        _gather_pos_body,
        out_type=(
            jax.ShapeDtypeStruct((E, 16), jnp.float32),
            jax.ShapeDtypeStruct((E, 16), jnp.float32),
        ),
        mesh=_mesh(),
        scratch_types=[
            pltpu.VMEM((CHUNK,), jnp.int32),
            pltpu.VMEM((CHUNK,), jnp.int32),
            pltpu.VMEM((CHUNK, 16), jnp.float32),
            pltpu.VMEM((CHUNK, 16), jnp.float32),
            pltpu.VMEM_SHARED((N, 16), jnp.float32),
            pltpu.SemaphoreType.DMA,
        ],
    )(pos16, sender, receiver)


def _gather_h_body(h_hbm, snd_hbm, hs_hbm, idx_s, rows, h_sh, sem):
    c = lax.axis_index("c")
    s = lax.axis_index("s")
    wid = s * NC + c

    @pl.when(s == 0)
    def _():
        pltpu.sync_copy(h_hbm, h_sh)

    plsc.subcore_barrier()
    nj = (NCHUNKS - wid + NW - 1) // NW

    def body(j, carry):
        base = (wid + j * NW) * CHUNK
        pltpu.sync_copy(snd_hbm.at[pl.ds(base, CHUNK)], idx_s)
        pltpu.async_copy(h_sh.at[idx_s], rows, sem).wait()
        pltpu.sync_copy(rows, hs_hbm.at[pl.ds(base, CHUNK)])
        return carry

    lax.fori_loop(0, nj, body, 0)


def _gather_h(h, sender):
    return pl.kernel(
        _gather_h_body,
        out_type=jax.ShapeDtypeStruct((E, H), jnp.float32),
        mesh=_mesh(),
        scratch_types=[
            pltpu.VMEM((CHUNK,), jnp.int32),
            pltpu.VMEM((CHUNK, H), jnp.float32),
            pltpu.VMEM_SHARED((N, H), jnp.float32),
            pltpu.SemaphoreType.DMA,
        ],
    )(h, sender)


NRB = 128
NBLOCKS = N // NRB
NTAIL = N - NBLOCKS * NRB


def _scatter_body(q_hbm, rcv_hbm, rnk_hbm, nr_hbm, zero_hbm, out_hbm,
                  idx_r, rnk, idx2, nrv, rows, acc, sem):
    c = lax.axis_index("c")
    s = lax.axis_index("s")
    nb = (NBLOCKS - s + NS - 1) // NS

    def zbody(b, carry):
        r = (s + b * NS) * NRB
        pltpu.sync_copy(zero_hbm, acc.at[pl.ds(r, NRB)])
        return carry

    lax.fori_loop(0, nb, zbody, 0)

    @pl.when(s == NS - 1)
    def _():
        pltpu.sync_copy(zero_hbm.at[pl.ds(0, NTAIL)],
                        acc.at[pl.ds(NBLOCKS * NRB, NTAIL)])

    plsc.subcore_barrier()

    half = NCHUNKS // NC
    lane = lax.broadcasted_iota(jnp.int32, (16,), 0)
    pltpu.sync_copy(nr_hbm, nrv.at[pl.ds(0, NCHUNKS)])
    gbase = N + s * CHUNK
    nj = (half - s + NS - 1) // NS

    def body(j, carry):
        ch = c * half + s + j * NS
        base = ch * CHUNK
        pltpu.sync_copy(rcv_hbm.at[pl.ds(base, CHUNK)], idx_r)
        pltpu.sync_copy(rnk_hbm.at[pl.ds(base, CHUNK)], rnk)
        pltpu.sync_copy(q_hbm.at[pl.ds(base, CHUNK)], rows)
        nrounds = nrv[pl.ds(ch, 16)][0]

        def rbody(r, c2):
            for v in range(CHUNK // 16):
                rv = rnk[pl.ds(v * 16, 16)]
                iv = idx_r[pl.ds(v * 16, 16)]
                garb = gbase + v * 16 + lane
                idx2[pl.ds(v * 16, 16)] = jnp.where(rv == r, iv, garb)
            pltpu.sync_copy(rows, acc.at[idx2], add=True)
            return c2

        lax.fori_loop(0, nrounds, rbody, 0)
        return carry

    lax.fori_loop(0, nj, body, 0)

    plsc.subcore_barrier()

    def obody(b, carry):
        r = (s + b * NS) * NRB
        pltpu.sync_copy(acc.at[pl.ds(r, NRB)], out_hbm.at[c, pl.ds(r, NRB)])
        return carry

    lax.fori_loop(0, nb, obody, 0)

    @pl.when(s == NS - 1)
    def _():
        pltpu.sync_copy(acc.at[pl.ds(NBLOCKS * NRB, NTAIL)],
                        out_hbm.at[c, pl.ds(NBLOCKS * NRB, NTAIL)])


def _sc_scatter(q, receiver, rank, nrounds, zeros_tile):
    return pl.kernel(
        _scatter_body,
        out_type=jax.ShapeDtypeStruct((NC, N, 2 * H), jnp.float32),
        mesh=_mesh(),
        scratch_types=[
            pltpu.VMEM((CHUNK,), jnp.int32),
            pltpu.VMEM((CHUNK,), jnp.int32),
            pltpu.VMEM((CHUNK,), jnp.int32),
            pltpu.VMEM((NCHUNKS + 16,), jnp.int32),
            pltpu.VMEM((CHUNK, 2 * H), jnp.float32),
            pltpu.VMEM_SHARED((N + NS * CHUNK, 2 * H), jnp.float32),
            pltpu.SemaphoreType.DMA,
        ],
    )(q, receiver, rank, nrounds, zeros_tile)


EB2 = 2560


def _rank_body(rcv_ref, rank_ref, nr_ref):
    row = lax.broadcasted_iota(jnp.int32, (CHUNK, CHUNK), 0)
    col = lax.broadcasted_iota(jnp.int32, (CHUNK, CHUNK), 1)
    earlier = col < row
    for c in range(EB2 // CHUNK):
        rc = rcv_ref[pl.ds(c * CHUNK, CHUNK), :]
        rt = jnp.reshape(rc, (1, CHUNK))
        eq = (rc == rt) & earlier
        cnt = jnp.sum(eq.astype(jnp.int32), axis=1, keepdims=True)
        rank_ref[pl.ds(c * CHUNK, CHUNK), :] = cnt
        nr_ref[0, pl.ds(c, 1), :] = jnp.max(cnt, axis=0, keepdims=True) + 1


def _tc_rank(receiver2d):
    return pl.pallas_call(
        _rank_body,
        grid=(E // EB2,),
        in_specs=[pl.BlockSpec((EB2, 1), lambda i: (i, 0))],
        out_specs=[
            pl.BlockSpec((EB2, 1), lambda i: (i, 0)),
            pl.BlockSpec((1, EB2 // CHUNK, 1), lambda i: (i, 0, 0)),
        ],
        out_shape=(
            jax.ShapeDtypeStruct((E, 1), jnp.int32),
            jax.ShapeDtypeStruct((E // EB2, EB2 // CHUNK, 1), jnp.int32),
        ),
    )(receiver2d)


def _edge_pre_body(ps_ref, pr_ref, shf_ref, wr10_ref, wr20_ref,
                   wr11_ref, wr21_ref, sh_ref, rw0_ref, rw1_ref):
    eps = 1e-9
    vec = pr_ref[...][:, :3] - ps_ref[...][:, :3] + shf_ref[...]
    r = jnp.sqrt(jnp.sum(vec * vec, axis=1, keepdims=True))
    v = vec / (r + eps)
    s3 = 3.0 ** 0.5
    ones = jnp.ones_like(r)
    sh_ref[...] = jnp.concatenate(
        [ones, s3 * v[:, 0:1], s3 * v[:, 1:2], s3 * v[:, 2:3]], axis=1)

    x = (jnp.pi / RMAX) * jnp.minimum(r, RMAX)
    uh = 0.5 * x

    def _cosp(t):
        t2 = t * t
        return 1.0 + t2 * (-0.5 + t2 * (1.0 / 24.0 + t2 * (-1.0 / 720.0
                   + t2 * (1.0 / 40320.0 + t2 * (-1.0 / 3628800.0)))))

    sin_h = _cosp(uh - (jnp.pi / 2))
    cos_h = _cosp(uh)
    s1 = 2.0 * sin_h * cos_h
    c1 = 1.0 - 2.0 * sin_h * sin_h
    u = r / RMAX
    u6 = u * u * u * u * u * u
    env = 1.0 - 28.0 * u6 + 48.0 * u6 * u - 21.0 * u6 * u * u
    env = env * (u < 1.0).astype(jnp.float32)
    w = (((2.0 / RMAX) ** 0.5) / (r + eps)) * env

    pre0 = jnp.zeros((EB, H), jnp.float32)
    pre1 = jnp.zeros((EB, H), jnp.float32)
    sprev = jnp.zeros_like(s1)
    scur = s1
    tc = 2.0 * c1
    for nn in range(NB):
        efn = w * scur
        pre0 = pre0 + efn * wr10_ref[nn:nn + 1, :]
        pre1 = pre1 + efn * wr11_ref[nn:nn + 1, :]
        snew = tc * scur - sprev
        sprev, scur = scur, snew

    rw0_ref[...] = jnp.dot(_silu(pre0), wr20_ref[...],
                           preferred_element_type=jnp.float32)
    rw1_ref[...] = jnp.dot(_silu(pre1), wr21_ref[...],
                           preferred_element_type=jnp.float32)


def _edge_pre(ps, pr, shifts, W_r1_0, W_r2_0, W_r1_1, W_r2_1):
    nsteps = E // EB
    wspec = pl.BlockSpec(index_map=lambda i: (0, 0))
    return pl.pallas_call(
        _edge_pre_body,
        grid=(nsteps,),
        in_specs=[
            pl.BlockSpec((EB, 16), lambda i: (i, 0)),
            pl.BlockSpec((EB, 16), lambda i: (i, 0)),
            pl.BlockSpec((EB, 3), lambda i: (i, 0)),
            wspec, wspec, wspec, wspec,
        ],
        out_specs=[
            pl.BlockSpec((EB, NSH), lambda i: (i, 0)),
            pl.BlockSpec((EB, H), lambda i: (i, 0)),
            pl.BlockSpec((EB, H), lambda i: (i, 0)),
        ],
        out_shape=(
            jax.ShapeDtypeStruct((E, NSH), jnp.float32),
            jax.ShapeDtypeStruct((E, H), jnp.float32),
            jax.ShapeDtypeStruct((E, H), jnp.float32),
        ),
    )(ps, pr, shifts, W_r1_0, W_r2_0, W_r1_1, W_r2_1)


def _msg_body(hs_ref, rw_ref, sh_ref, wd_ref, out_ref):
    ev = hs_ref[...] * rw_ref[...]
    sh = sh_ref[...]
    wd = wd_ref[...]
    q = jnp.zeros((EB, H), jnp.float32)
    for s in range(NSH):
        t = jnp.dot(ev, wd[s * H:(s + 1) * H, :],
                    preferred_element_type=jnp.float32)
        q = q + t * sh[:, s:s + 1]
    out_ref[:, :H] = q
    out_ref[:, H:] = jnp.zeros((EB, H), jnp.float32)


def _tc_msg(hs, rw, sh, W_down):
    nsteps = E // EB
    return pl.pallas_call(
        _msg_body,
        grid=(nsteps,),
        in_specs=[
            pl.BlockSpec((EB, H), lambda i: (i, 0)),
            pl.BlockSpec((EB, H), lambda i: (i, 0)),
            pl.BlockSpec((EB, NSH), lambda i: (i, 0)),
            pl.BlockSpec(index_map=lambda i: (0, 0)),
        ],
        out_specs=pl.BlockSpec((EB, 2 * H), lambda i: (i, 0)),
        out_shape=jax.ShapeDtypeStruct((E, 2 * H), jnp.float32),
    )(hs, rw, sh, W_down)


def _layer_body(a0_ref, a1_ref, nf_ref, b_ref, eprev_ref,
                wsc_ref, wp_ref, wread_ref, wupn_ref,
                nf2_ref, h_ref, e_ref):
    msg = a0_ref[0][:, :H] + a1_ref[0][:, :H]
    nf2 = (jnp.dot(_silu(msg), wp_ref[...], preferred_element_type=jnp.float32)
           + jnp.dot(nf_ref[...], wsc_ref[...], preferred_element_type=jnp.float32))
    nf2_ref[...] = nf2
    h_ref[...] = jnp.dot(nf2, wupn_ref[...], preferred_element_type=jnp.float32)
    es = jnp.dot(nf2, wread_ref[...], preferred_element_type=jnp.float32)
    onehot = (b_ref[...] == lax.broadcasted_iota(jnp.int32, (1, G), 1)
              ).astype(jnp.float32)
    contrib = jnp.sum(onehot * es, axis=0, keepdims=True)

    @pl.when(pl.program_id(0) == 0)
    def _():
        e_ref[...] = eprev_ref[...] + contrib

    @pl.when(pl.program_id(0) > 0)
    def _():
        e_ref[...] = e_ref[...] + contrib


def _tc_layer(agg, nf, batch2d, e_prev, W_sc, W_prod, W_read, W_up_next):
    nsteps = N // NBLK
    wspec = pl.BlockSpec(index_map=lambda i: (0, 0))
    return pl.pallas_call(
        _layer_body,
        grid=(nsteps,),
        in_specs=[
            pl.BlockSpec((1, NBLK, 2 * H), lambda i: (0, i, 0)),
            pl.BlockSpec((1, NBLK, 2 * H), lambda i: (1, i, 0)),
            pl.BlockSpec((NBLK, H), lambda i: (i, 0)),
            pl.BlockSpec((NBLK, 1), lambda i: (i, 0)),
            wspec, wspec, wspec, wspec, wspec,
        ],
        out_specs=[
            pl.BlockSpec((NBLK, H), lambda i: (i, 0)),
            pl.BlockSpec((NBLK, H), lambda i: (i, 0)),
            pl.BlockSpec((1, G), lambda i: (0, 0)),
        ],
        out_shape=(
            jax.ShapeDtypeStruct((N, H), jnp.float32),
            jax.ShapeDtypeStruct((N, H), jnp.float32),
            jax.ShapeDtypeStruct((1, G), jnp.float32),
        ),
    )(agg, agg, nf, batch2d, e_prev, W_sc, W_prod, W_read, W_up_next)


def kernel(positions, node_attrs, edge_index, batch, head, shifts, ptr,
           W_ae, W_emb, W_up0, W_r1_0, W_r2_0, W_down0, W_sc0, W_prod0,
           W_read0, W_up1, W_r1_1, W_r2_1, W_down1, W_sc1, W_prod1, W_read1):
    sender = edge_index[0].astype(jnp.int32)
    receiver = edge_index[1].astype(jnp.int32)
    pos16 = jnp.pad(positions, ((0, 0), (0, 13)))
    batch2d = batch.reshape(N, 1).astype(jnp.int32)
    zeros_tile = jnp.zeros((NRB, 2 * H), jnp.float32)

    nf, h, e = _node_init(node_attrs, batch2d, W_ae, W_emb, W_up0)
    ps, pr = _gather_pos(pos16, sender, receiver)
    sh, rw0, rw1 = _edge_pre(ps, pr, shifts, W_r1_0, W_r2_0, W_r1_1, W_r2_1)
    rank2d, nrounds2d = _tc_rank(receiver.reshape(E, 1))
    rank = rank2d.reshape(E)
    nrounds = nrounds2d.reshape(NCHUNKS)

    layer_params = [
        (rw0, W_down0, W_sc0, W_prod0, W_read0, W_up1),
        (rw1, W_down1, W_sc1, W_prod1, W_read1, W_up1),
    ]
    for rw, W_down, W_sc, W_prod, W_read, W_up_next in layer_params:
        hs = _gather_h(h, sender)
        q = _tc_msg(hs, rw, sh, W_down)
        agg = _sc_scatter(q, receiver, rank, nrounds, zeros_tile)
        nf, h, e = _tc_layer(agg, nf, batch2d, e, W_sc, W_prod,
                             W_read, W_up_next)
    return e.reshape(G)
